# baseline (speedup 1.0000x reference)
import functools

import jax
import jax.numpy as jnp
import numpy as np
from jax import lax
from jax.experimental import pallas as pl
from jax.experimental.pallas import tpu as pltpu
from jax.experimental.pallas import tpu_sc as plsc

_N = 10000
_E = 320000
_D = 128
_DE = 16
_H = 128

_NC = 2
_NS = 16
_NW = _NC * _NS
_L = 16

_CH = 64
_CPT = 160
_DBLK = 8
_BPT = _CPT // _DBLK
_EP = _NW * _CPT * _CH
_NP = 10112
_RPS = _NP // _NS
_NPC = 10240
_CPS = _NPC // _NS

_NBLK = 1000
_EBLK = 2048

_COLA = np.array([32 * (t // 16) + t % 16 for t in range(_D // 2)],
                 dtype=np.int32)
_COLB = _COLA + 16


def _node_pre_body(x_ref, w_ref, oa_ref, ob_ref, osk_ref):
    xw = jnp.dot(x_ref[...], w_ref[...],
                 preferred_element_type=jnp.float32)
    oa_ref[...] = xw[:, :_H]
    ob_ref[...] = xw[:, _H:2 * _H]
    osk_ref[...] = xw[:, 2 * _H:]


def _edge_pre_body(ea_ref, wa_ref, wb_ref, ba_ref, bb_ref, o_ref):
    ea = ea_ref[...]
    oa = jnp.dot(ea, wa_ref[...],
                 preferred_element_type=jnp.float32) + ba_ref[...]
    ob = jnp.dot(ea, wb_ref[...],
                 preferred_element_type=jnp.float32) + bb_ref[...]
    ua = lax.bitcast_convert_type(oa.astype(jnp.bfloat16),
                                  jnp.uint16).astype(jnp.uint32)
    ub = lax.bitcast_convert_type(ob.astype(jnp.bfloat16),
                                  jnp.uint16).astype(jnp.uint32)
    o_ref[...] = lax.bitcast_convert_type(ua | (ub << 16), jnp.int32)


def _post_body(s_ref, cnt_ref, skip_ref, w2_ref, b2_ref, g_ref, be_ref,
               o_ref):
    s = s_ref[0] + s_ref[1]
    cnt = cnt_ref[:, 0:1] + cnt_ref[:, 1:2]
    agg = jnp.dot(s, w2_ref[...], preferred_element_type=jnp.float32)
    agg = (agg + cnt * b2_ref[...]) / jnp.maximum(cnt, 1.0)
    out = agg + skip_ref[...]
    mu = jnp.mean(out, axis=-1, keepdims=True)
    var = jnp.mean((out - mu) ** 2, axis=-1, keepdims=True)
    o_ref[...] = (out - mu) * lax.rsqrt(var + 1e-5) * g_ref[...] + be_ref[...]


def _sc_edge_body(dst_hbm, src_hbm, c_hbm, a_hbm, b_hbm, z2_hbm, z1_hbm,
                  out_s_hbm, out_c_hbm,
                  dblk0, sblk0, dblk1, sblk1, av0, bv0, av1, bv1, cv0,
                  onesv, s_sh, cnt_sh,
                  sema0, semb0, sema1, semb1, semc0, semcnt):
    cid = lax.axis_index("c")
    sid = lax.axis_index("s")
    tbase = (sid * _NC + cid) * _CPT

    my_rows = pl.ds(sid * _RPS, _RPS)
    pltpu.sync_copy(z2_hbm, s_sh.at[my_rows])
    pltpu.sync_copy(z1_hbm, cnt_sh.at[pl.ds(sid * _CPS, _CPS)])

    for i in range(_CH // _L):
        onesv[pl.ds(i * _L, _L)] = jnp.full((_L,), 1.0, jnp.float32)

    plsc.subcore_barrier()

    gsets = ((av0, bv0, sema0, semb0),
             (av1, bv1, sema1, semb1))
    iblks = ((dblk0, sblk0), (dblk1, sblk1))

    def load_iblk(b, ib):
        d_, s_ = ib
        r0 = tbase + b * _DBLK
        pltpu.sync_copy(dst_hbm.at[pl.ds(r0, _DBLK)], d_)
        pltpu.sync_copy(src_hbm.at[pl.ds(r0, _DBLK)], s_)

    def fetch(idx_d, idx_s, gs):
        av, bv, sa, sb = gs
        pltpu.async_copy(a_hbm.at[idx_d], av, sa)
        pltpu.async_copy(b_hbm.at[idx_s], bv, sb)

    def process(g, idx_d, gs):
        av, bv, sa, sb = gs
        cv = cv0
        pltpu.make_async_copy(a_hbm.at[idx_d], av, sa).wait()
        pltpu.make_async_copy(b_hbm.at[idx_d], bv, sb).wait()
        pltpu.make_async_copy(c_hbm.at[pl.ds(0, _CH)], cv, semc0).wait()

        sh16 = jnp.full((_L,), 16, jnp.int32)
        msk = jnp.full((_L,), -65536, jnp.int32)

        @plsc.parallel_loop(0, _CH, unroll=4)
        def _(r):
            for j in range(_D // 32):
                v = cv[r, pl.ds(_L * j, _L)]
                c_lo = lax.bitcast_convert_type(
                    lax.shift_left(v, sh16), jnp.float32)
                c_hi = lax.bitcast_convert_type(
                    lax.bitwise_and(v, msk), jnp.float32)
                slo = pl.ds(32 * j, _L)
                shi = pl.ds(32 * j + _L, _L)
                av[r, slo] = jnp.maximum(av[r, slo] + bv[r, slo] + c_lo, 0.0)
                av[r, shi] = jnp.maximum(av[r, shi] + bv[r, shi] + c_hi, 0.0)

        pltpu.async_copy(
            c_hbm.at[pl.ds((tbase + jnp.minimum(g + 1, _CPT - 1)) * _CH,
                           _CH)], cv, semc0)
        pltpu.sync_copy(av, s_sh.at[idx_d], add=True)
        pltpu.async_copy(onesv, cnt_sh.at[idx_d], semcnt, add=True)

    def drain_cnt():
        for _ in range(_DBLK):
            pltpu.make_async_copy(onesv, cnt_sh.at[dblk0.at[0]],
                                  semcnt).wait()

    def block(b, cur, other):
        d_, s_ = cur
        do_, so_ = other

        @pl.when(b > 0)
        def _():
            drain_cnt()

        load_iblk(jnp.minimum(b + 1, _BPT - 1), other)

        for j in range(_DBLK):
            g = b * _DBLK + j
            p = j % 2
            if j < _DBLK - 1:
                fetch(d_.at[j + 1], s_.at[j + 1], gsets[1 - p])
            else:
                fetch(do_.at[0], so_.at[0], gsets[1 - p])
            process(g, d_.at[j], gsets[p])

    load_iblk(0, iblks[0])
    fetch(dblk0.at[0], sblk0.at[0], gsets[0])
    pltpu.async_copy(c_hbm.at[pl.ds(tbase * _CH, _CH)], cv0, semc0)

    def super_body(i, carry):
        block(2 * i, iblks[0], iblks[1])
        block(2 * i + 1, iblks[1], iblks[0])
        return carry

    lax.fori_loop(0, _BPT // 2, super_body, 0)

    av, bv, sa, sb = gsets[0]
    pltpu.make_async_copy(a_hbm.at[dblk0.at[0]], av, sa).wait()
    pltpu.make_async_copy(b_hbm.at[dblk0.at[0]], bv, sb).wait()
    pltpu.make_async_copy(c_hbm.at[pl.ds(0, _CH)], cv0, semc0).wait()
    drain_cnt()

    plsc.subcore_barrier()
    pltpu.sync_copy(s_sh.at[my_rows], out_s_hbm.at[cid, my_rows])

    @pl.when(sid == 0)
    def _():
        pltpu.sync_copy(cnt_sh, out_c_hbm.at[cid, 0])


def _build(interpret=False):
    node_pre = pl.pallas_call(
        _node_pre_body,
        grid=(_NS,),
        in_specs=[
            pl.BlockSpec((_RPS, _D), lambda i: (i, 0)),
            pl.BlockSpec((_D, 3 * _H), lambda i: (0, 0)),
        ],
        out_specs=[
            pl.BlockSpec((_RPS, _H), lambda i: (i, 0)),
            pl.BlockSpec((_RPS, _H), lambda i: (i, 0)),
            pl.BlockSpec((_RPS, _H), lambda i: (i, 0)),
        ],
        out_shape=[
            jax.ShapeDtypeStruct((_NP, _H), jnp.float32),
            jax.ShapeDtypeStruct((_NP, _H), jnp.float32),
            jax.ShapeDtypeStruct((_NP, _H), jnp.float32),
        ],
        interpret=interpret,
    )

    edge_pre = pl.pallas_call(
        _edge_pre_body,
        grid=(_EP // _EBLK,),
        in_specs=[
            pl.BlockSpec((_EBLK, _DE), lambda i: (i, 0)),
            pl.BlockSpec((_DE, _H // 2), lambda i: (0, 0)),
            pl.BlockSpec((_DE, _H // 2), lambda i: (0, 0)),
            pl.BlockSpec((1, _H // 2), lambda i: (0, 0)),
            pl.BlockSpec((1, _H // 2), lambda i: (0, 0)),
        ],
        out_specs=pl.BlockSpec((_EBLK, _H // 2), lambda i: (i, 0)),
        out_shape=jax.ShapeDtypeStruct((_EP, _H // 2), jnp.int32),
        interpret=interpret,
    )

    mesh = plsc.VectorSubcoreMesh(core_axis_name="c", subcore_axis_name="s")
    sc_edge = pl.kernel(
        _sc_edge_body,
        out_type=(
            jax.ShapeDtypeStruct((_NC, _NP, _D), jnp.float32),
            jax.ShapeDtypeStruct((_NC, 1, _NPC), jnp.float32),
        ),
        mesh=mesh,
        scratch_types=[
            pltpu.VMEM((_DBLK, _CH), jnp.int32),
            pltpu.VMEM((_DBLK, _CH), jnp.int32),
            pltpu.VMEM((_DBLK, _CH), jnp.int32),
            pltpu.VMEM((_DBLK, _CH), jnp.int32),
            pltpu.VMEM((_CH, _D), jnp.float32),
            pltpu.VMEM((_CH, _D), jnp.float32),
            pltpu.VMEM((_CH, _D), jnp.float32),
            pltpu.VMEM((_CH, _D), jnp.float32),
            pltpu.VMEM((_CH, _D // 2), jnp.int32),
            pltpu.VMEM((_CH,), jnp.float32),
            pltpu.VMEM_SHARED((_NP, _D), jnp.float32),
            pltpu.VMEM_SHARED((_NPC,), jnp.float32),
            pltpu.SemaphoreType.DMA,
            pltpu.SemaphoreType.DMA,
            pltpu.SemaphoreType.DMA,
            pltpu.SemaphoreType.DMA,
            pltpu.SemaphoreType.DMA,
            pltpu.SemaphoreType.DMA,
        ],
        interpret=interpret,
    )

    post = pl.pallas_call(
        _post_body,
        grid=(_N // _NBLK,),
        in_specs=[
            pl.BlockSpec((_NC, _NBLK, _D), lambda i: (0, i, 0)),
            pl.BlockSpec((_NBLK, _NC), lambda i: (i, 0)),
            pl.BlockSpec((_NBLK, _D), lambda i: (i, 0)),
            pl.BlockSpec((_D, _D), lambda i: (0, 0)),
            pl.BlockSpec((1, _D), lambda i: (0, 0)),
            pl.BlockSpec((1, _D), lambda i: (0, 0)),
            pl.BlockSpec((1, _D), lambda i: (0, 0)),
        ],
        out_specs=pl.BlockSpec((_NBLK, _D), lambda i: (i, 0)),
        out_shape=jax.ShapeDtypeStruct((_N, _D), jnp.float32),
        interpret=interpret,
    )

    @jax.jit
    def run(x, edge_index, edge_attr, W1, b1, W2, b2, Wskip, gamma, beta):
        dst = edge_index[0]
        src = edge_index[1]

        wn = jnp.concatenate([W1[:_D], W1[_D:2 * _D], Wskip], axis=1)
        a_tab, b_tab, skip = node_pre(x, wn)

        pad_dst = _N + (jnp.arange(_EP - _E, dtype=jnp.int32) % (_NP - _N))
        dst_p = jnp.concatenate([dst, pad_dst]).reshape(_EP // _CH, _CH)
        src_p = jnp.concatenate(
            [src, jnp.zeros((_EP - _E,), jnp.int32)]).reshape(
                _EP // _CH, _CH)
        ea_p = jnp.pad(edge_attr, ((0, _EP - _E), (0, 0)))
        w1e = W1[2 * _D:]
        cola = jnp.asarray(_COLA)
        colb = jnp.asarray(_COLB)
        c_i32 = edge_pre(ea_p, w1e[:, cola], w1e[:, colb],
                         b1[cola].reshape(1, _H // 2),
                         b1[colb].reshape(1, _H // 2))

        z2 = jnp.zeros((_RPS, _D), jnp.float32)
        z1 = jnp.zeros((_CPS,), jnp.float32)
        s_out, cnt_out = sc_edge(dst_p, src_p, c_i32, a_tab, b_tab, z2, z1)

        cnt_t = jnp.swapaxes(cnt_out.reshape(_NC, _NPC), 0, 1)
        out = post(s_out, cnt_t, skip, W2, b2.reshape(1, _D),
                   gamma.reshape(1, _D), beta.reshape(1, _D))
        return out

    return run


_impl = _build()


def kernel(x, edge_index, edge_attr, W1, b1, W2, b2, Wskip, gamma, beta):
    return _impl(x, edge_index, edge_attr, W1, b1, W2, b2, Wskip, gamma,
                 beta)


# speedup vs baseline: 1.2367x; 1.2367x over previous
import functools

import jax
import jax.numpy as jnp
import numpy as np
from jax import lax
from jax.experimental import pallas as pl
from jax.experimental.pallas import tpu as pltpu
from jax.experimental.pallas import tpu_sc as plsc

_N = 10000
_E = 320000
_D = 128
_DE = 16
_H = 128

_NC = 2
_NS = 16
_NW = _NC * _NS
_L = 16

_CH = 64
_CPT = 160
_DBLK = 8
_BPT = _CPT // _DBLK
_EP = _NW * _CPT * _CH
_NP = 10112
_RPS = _NP // _NS
_NPC = 10240
_CPS = _NPC // _NS

_NBLK = 1000
_EBLK = 8192

_COLA = np.array([32 * (t // 16) + t % 16 for t in range(_D // 2)],
                 dtype=np.int32)
_COLB = _COLA + 16


def _node_pre_body(x_ref, w_ref, oa_ref, ob_ref, osk_ref):
    xw = jnp.dot(x_ref[...], w_ref[...],
                 preferred_element_type=jnp.float32)
    oa_ref[...] = xw[:, :_H]
    ob_ref[...] = xw[:, _H:2 * _H]
    osk_ref[...] = xw[:, 2 * _H:]


def _edge_pre_body(ea_ref, w_ref, b_ref, o_ref):
    o = jnp.dot(ea_ref[...], w_ref[...],
                preferred_element_type=jnp.float32) + b_ref[...]
    oa = o[:, :_H // 2]
    ob = o[:, _H // 2:]
    ua = lax.bitcast_convert_type(oa.astype(jnp.bfloat16),
                                  jnp.uint16).astype(jnp.uint32)
    ub = lax.bitcast_convert_type(ob.astype(jnp.bfloat16),
                                  jnp.uint16).astype(jnp.uint32)
    o_ref[...] = lax.bitcast_convert_type(ua | (ub << 16), jnp.int32)


def _post_body(s_ref, cnt_ref, skip_ref, w2_ref, b2_ref, g_ref, be_ref,
               o_ref):
    s = s_ref[0] + s_ref[1]
    cnt = cnt_ref[:, 0:1] + cnt_ref[:, 1:2]
    agg = jnp.dot(s, w2_ref[...], preferred_element_type=jnp.float32)
    agg = (agg + cnt * b2_ref[...]) / jnp.maximum(cnt, 1.0)
    out = agg + skip_ref[...]
    mu = jnp.mean(out, axis=-1, keepdims=True)
    var = jnp.mean((out - mu) ** 2, axis=-1, keepdims=True)
    o_ref[...] = (out - mu) * lax.rsqrt(var + 1e-5) * g_ref[...] + be_ref[...]


def _sc_edge_body(dst_hbm, src_hbm, c_hbm, a_hbm, b_hbm, z2_hbm, z1_hbm,
                  out_s_hbm, out_c_hbm,
                  dblk0, sblk0, dblk1, sblk1, av0, bv0, av1, bv1, cv0,
                  onesv, s_sh, cnt_sh,
                  sema0, semb0, sema1, semb1, semc0, semcnt):
    cid = lax.axis_index("c")
    sid = lax.axis_index("s")
    tbase = (sid * _NC + cid) * _CPT

    my_rows = pl.ds(sid * _RPS, _RPS)
    pltpu.sync_copy(z2_hbm, s_sh.at[my_rows])
    pltpu.sync_copy(z1_hbm, cnt_sh.at[pl.ds(sid * _CPS, _CPS)])

    for i in range(_CH // _L):
        onesv[pl.ds(i * _L, _L)] = jnp.full((_L,), 1.0, jnp.float32)

    plsc.subcore_barrier()

    gsets = ((av0, bv0, sema0, semb0),
             (av1, bv1, sema1, semb1))
    iblks = ((dblk0, sblk0), (dblk1, sblk1))

    def load_iblk(b, ib):
        d_, s_ = ib
        r0 = tbase + b * _DBLK
        pltpu.sync_copy(dst_hbm.at[pl.ds(r0, _DBLK)], d_)
        pltpu.sync_copy(src_hbm.at[pl.ds(r0, _DBLK)], s_)

    def fetch(idx_d, idx_s, gs):
        av, bv, sa, sb = gs
        pltpu.async_copy(a_hbm.at[idx_d], av, sa)
        pltpu.async_copy(b_hbm.at[idx_s], bv, sb)

    def process(g, idx_d, gs):
        av, bv, sa, sb = gs
        cv = cv0
        pltpu.make_async_copy(a_hbm.at[idx_d], av, sa).wait()
        pltpu.make_async_copy(b_hbm.at[idx_d], bv, sb).wait()
        pltpu.make_async_copy(c_hbm.at[pl.ds(0, _CH)], cv, semc0).wait()

        sh16 = jnp.full((_L,), 16, jnp.int32)
        msk = jnp.full((_L,), -65536, jnp.int32)

        @plsc.parallel_loop(0, _CH, unroll=4)
        def _(r):
            for j in range(_D // 32):
                v = cv[r, pl.ds(_L * j, _L)]
                c_lo = lax.bitcast_convert_type(
                    lax.shift_left(v, sh16), jnp.float32)
                c_hi = lax.bitcast_convert_type(
                    lax.bitwise_and(v, msk), jnp.float32)
                slo = pl.ds(32 * j, _L)
                shi = pl.ds(32 * j + _L, _L)
                av[r, slo] = jnp.maximum(av[r, slo] + bv[r, slo] + c_lo, 0.0)
                av[r, shi] = jnp.maximum(av[r, shi] + bv[r, shi] + c_hi, 0.0)

        pltpu.async_copy(
            c_hbm.at[pl.ds((tbase + jnp.minimum(g + 1, _CPT - 1)) * _CH,
                           _CH)], cv, semc0)
        pltpu.sync_copy(av, s_sh.at[idx_d], add=True)
        pltpu.async_copy(onesv, cnt_sh.at[idx_d], semcnt, add=True)

    def drain_cnt():
        for _ in range(_DBLK):
            pltpu.make_async_copy(onesv, cnt_sh.at[dblk0.at[0]],
                                  semcnt).wait()

    def block(b, cur, other):
        d_, s_ = cur
        do_, so_ = other

        @pl.when(b > 0)
        def _():
            drain_cnt()

        load_iblk(jnp.minimum(b + 1, _BPT - 1), other)

        for j in range(_DBLK):
            g = b * _DBLK + j
            p = j % 2
            if j < _DBLK - 1:
                fetch(d_.at[j + 1], s_.at[j + 1], gsets[1 - p])
            else:
                fetch(do_.at[0], so_.at[0], gsets[1 - p])
            process(g, d_.at[j], gsets[p])

    load_iblk(0, iblks[0])
    fetch(dblk0.at[0], sblk0.at[0], gsets[0])
    pltpu.async_copy(c_hbm.at[pl.ds(tbase * _CH, _CH)], cv0, semc0)

    def super_body(i, carry):
        block(2 * i, iblks[0], iblks[1])
        block(2 * i + 1, iblks[1], iblks[0])
        return carry

    lax.fori_loop(0, _BPT // 2, super_body, 0)

    av, bv, sa, sb = gsets[0]
    pltpu.make_async_copy(a_hbm.at[dblk0.at[0]], av, sa).wait()
    pltpu.make_async_copy(b_hbm.at[dblk0.at[0]], bv, sb).wait()
    pltpu.make_async_copy(c_hbm.at[pl.ds(0, _CH)], cv0, semc0).wait()
    drain_cnt()

    plsc.subcore_barrier()
    pltpu.sync_copy(s_sh.at[my_rows], out_s_hbm.at[cid, my_rows])

    @pl.when(sid == 0)
    def _():
        pltpu.sync_copy(cnt_sh, out_c_hbm.at[cid, 0])


def _build(interpret=False):
    node_pre = pl.pallas_call(
        _node_pre_body,
        grid=(_NS,),
        in_specs=[
            pl.BlockSpec((_RPS, _D), lambda i: (i, 0)),
            pl.BlockSpec((_D, 3 * _H), lambda i: (0, 0)),
        ],
        out_specs=[
            pl.BlockSpec((_RPS, _H), lambda i: (i, 0)),
            pl.BlockSpec((_RPS, _H), lambda i: (i, 0)),
            pl.BlockSpec((_RPS, _H), lambda i: (i, 0)),
        ],
        out_shape=[
            jax.ShapeDtypeStruct((_NP, _H), jnp.float32),
            jax.ShapeDtypeStruct((_NP, _H), jnp.float32),
            jax.ShapeDtypeStruct((_NP, _H), jnp.float32),
        ],
        interpret=interpret,
    )

    edge_pre = pl.pallas_call(
        _edge_pre_body,
        grid=(_EP // _EBLK,),
        in_specs=[
            pl.BlockSpec((_EBLK, _DE), lambda i: (i, 0)),
            pl.BlockSpec((_DE, _H), lambda i: (0, 0)),
            pl.BlockSpec((1, _H), lambda i: (0, 0)),
        ],
        out_specs=pl.BlockSpec((_EBLK, _H // 2), lambda i: (i, 0)),
        out_shape=jax.ShapeDtypeStruct((_EP, _H // 2), jnp.int32),
        interpret=interpret,
    )

    mesh = plsc.VectorSubcoreMesh(core_axis_name="c", subcore_axis_name="s")
    sc_edge = pl.kernel(
        _sc_edge_body,
        out_type=(
            jax.ShapeDtypeStruct((_NC, _NP, _D), jnp.float32),
            jax.ShapeDtypeStruct((_NC, 1, _NPC), jnp.float32),
        ),
        mesh=mesh,
        scratch_types=[
            pltpu.VMEM((_DBLK, _CH), jnp.int32),
            pltpu.VMEM((_DBLK, _CH), jnp.int32),
            pltpu.VMEM((_DBLK, _CH), jnp.int32),
            pltpu.VMEM((_DBLK, _CH), jnp.int32),
            pltpu.VMEM((_CH, _D), jnp.float32),
            pltpu.VMEM((_CH, _D), jnp.float32),
            pltpu.VMEM((_CH, _D), jnp.float32),
            pltpu.VMEM((_CH, _D), jnp.float32),
            pltpu.VMEM((_CH, _D // 2), jnp.int32),
            pltpu.VMEM((_CH,), jnp.float32),
            pltpu.VMEM_SHARED((_NP, _D), jnp.float32),
            pltpu.VMEM_SHARED((_NPC,), jnp.float32),
            pltpu.SemaphoreType.DMA,
            pltpu.SemaphoreType.DMA,
            pltpu.SemaphoreType.DMA,
            pltpu.SemaphoreType.DMA,
            pltpu.SemaphoreType.DMA,
            pltpu.SemaphoreType.DMA,
        ],
        interpret=interpret,
    )

    post = pl.pallas_call(
        _post_body,
        grid=(_N // _NBLK,),
        in_specs=[
            pl.BlockSpec((_NC, _NBLK, _D), lambda i: (0, i, 0)),
            pl.BlockSpec((_NBLK, _NC), lambda i: (i, 0)),
            pl.BlockSpec((_NBLK, _D), lambda i: (i, 0)),
            pl.BlockSpec((_D, _D), lambda i: (0, 0)),
            pl.BlockSpec((1, _D), lambda i: (0, 0)),
            pl.BlockSpec((1, _D), lambda i: (0, 0)),
            pl.BlockSpec((1, _D), lambda i: (0, 0)),
        ],
        out_specs=pl.BlockSpec((_NBLK, _D), lambda i: (i, 0)),
        out_shape=jax.ShapeDtypeStruct((_N, _D), jnp.float32),
        interpret=interpret,
    )

    @jax.jit
    def run(x, edge_index, edge_attr, W1, b1, W2, b2, Wskip, gamma, beta):
        dst = edge_index[0]
        src = edge_index[1]

        wn = jnp.concatenate([W1[:_D], W1[_D:2 * _D], Wskip], axis=1)
        a_tab, b_tab, skip = node_pre(x, wn)

        pad_dst = _N + (jnp.arange(_EP - _E, dtype=jnp.int32) % (_NP - _N))
        dst_p = jnp.concatenate([dst, pad_dst]).reshape(_EP // _CH, _CH)
        src_p = jnp.concatenate(
            [src, jnp.zeros((_EP - _E,), jnp.int32)]).reshape(
                _EP // _CH, _CH)
        ea_p = jnp.pad(edge_attr, ((0, _EP - _E), (0, 0)))
        w1e = W1[2 * _D:]
        colab = jnp.concatenate([jnp.asarray(_COLA), jnp.asarray(_COLB)])
        c_i32 = edge_pre(ea_p, w1e[:, colab], b1[colab].reshape(1, _H))

        z2 = jnp.zeros((_RPS, _D), jnp.float32)
        z1 = jnp.zeros((_CPS,), jnp.float32)
        s_out, cnt_out = sc_edge(dst_p, src_p, c_i32, a_tab, b_tab, z2, z1)

        cnt_t = jnp.swapaxes(cnt_out.reshape(_NC, _NPC), 0, 1)
        out = post(s_out, cnt_t, skip, W2, b2.reshape(1, _D),
                   gamma.reshape(1, _D), beta.reshape(1, _D))
        return out

    return run


_impl = _build()


def kernel(x, edge_index, edge_attr, W1, b1, W2, b2, Wskip, gamma, beta):
    return _impl(x, edge_index, edge_attr, W1, b1, W2, b2, Wskip, gamma,
                 beta)


# speedup vs baseline: 1.2705x; 1.0273x over previous
import functools

import jax
import jax.numpy as jnp
import numpy as np
from jax import lax
from jax.experimental import pallas as pl
from jax.experimental.pallas import tpu as pltpu
from jax.experimental.pallas import tpu_sc as plsc

_N = 10000
_E = 320000
_D = 128
_DE = 16
_H = 128

_NC = 2
_NS = 16
_NW = _NC * _NS
_L = 16

_CH = 64
_CPT = 160
_K0 = 192
_K1 = 2 * _CPT - _K0
_DBLK = 8
_EP = _NW * _CPT * _CH
_NP = 10112
_RPS = _NP // _NS
_NPC = 10240
_CPS = _NPC // _NS

_NBLK = 1000
_EBLK = 8192

_COLA = np.array([32 * (t // 16) + t % 16 for t in range(_D // 2)],
                 dtype=np.int32)
_COLB = _COLA + 16


def _node_pre_body(x_ref, w_ref, oa_ref, ob_ref, osk_ref):
    xw = jnp.dot(x_ref[...], w_ref[...],
                 preferred_element_type=jnp.float32)
    oa_ref[...] = xw[:, :_H]
    ob_ref[...] = xw[:, _H:2 * _H]
    osk_ref[...] = xw[:, 2 * _H:]


def _edge_pre_body(ea_ref, w_ref, b_ref, o_ref):
    o = jnp.dot(ea_ref[...], w_ref[...],
                preferred_element_type=jnp.float32) + b_ref[...]
    oa = o[:, :_H // 2]
    ob = o[:, _H // 2:]
    ua = lax.bitcast_convert_type(oa.astype(jnp.bfloat16),
                                  jnp.uint16).astype(jnp.uint32)
    ub = lax.bitcast_convert_type(ob.astype(jnp.bfloat16),
                                  jnp.uint16).astype(jnp.uint32)
    o_ref[...] = lax.bitcast_convert_type(ua | (ub << 16), jnp.int32)


def _post_body(s_ref, cnt_ref, skip_ref, w2_ref, b2_ref, g_ref, be_ref,
               o_ref):
    s = s_ref[0] + s_ref[1]
    cnt = cnt_ref[:, 0:1] + cnt_ref[:, 1:2]
    agg = jnp.dot(s, w2_ref[...], preferred_element_type=jnp.float32)
    agg = (agg + cnt * b2_ref[...]) / jnp.maximum(cnt, 1.0)
    out = agg + skip_ref[...]
    mu = jnp.mean(out, axis=-1, keepdims=True)
    var = jnp.mean((out - mu) ** 2, axis=-1, keepdims=True)
    o_ref[...] = (out - mu) * lax.rsqrt(var + 1e-5) * g_ref[...] + be_ref[...]


def _sc_edge_body(dst_hbm, src_hbm, c_hbm, a_hbm, b_hbm, z2_hbm, z1_hbm,
                  out_s_hbm, out_c_hbm,
                  dblk0, sblk0, dblk1, sblk1, av0, bv0, av1, bv1, cv0,
                  onesv, s_sh, cnt_sh,
                  sema0, semb0, sema1, semb1, semc0, semcnt):
    cid = lax.axis_index("c")
    sid = lax.axis_index("s")
    kc = lax.select(cid == 0, _K0, _K1)
    nblk = kc // _DBLK
    tbase = lax.select(cid == 0, sid * _K0, _NS * _K0 + sid * _K1)

    my_rows = pl.ds(sid * _RPS, _RPS)
    pltpu.sync_copy(z2_hbm, s_sh.at[my_rows])
    pltpu.sync_copy(z1_hbm, cnt_sh.at[pl.ds(sid * _CPS, _CPS)])

    for i in range(_CH // _L):
        onesv[pl.ds(i * _L, _L)] = jnp.full((_L,), 1.0, jnp.float32)

    plsc.subcore_barrier()

    gsets = ((av0, bv0, sema0, semb0),
             (av1, bv1, sema1, semb1))
    iblks = ((dblk0, sblk0), (dblk1, sblk1))

    def load_iblk(b, ib):
        d_, s_ = ib
        r0 = tbase + b * _DBLK
        pltpu.sync_copy(dst_hbm.at[pl.ds(r0, _DBLK)], d_)
        pltpu.sync_copy(src_hbm.at[pl.ds(r0, _DBLK)], s_)

    def fetch(idx_d, idx_s, gs):
        av, bv, sa, sb = gs
        pltpu.async_copy(a_hbm.at[idx_d], av, sa)
        pltpu.async_copy(b_hbm.at[idx_s], bv, sb)

    def process(g, idx_d, gs):
        av, bv, sa, sb = gs
        cv = cv0
        pltpu.make_async_copy(a_hbm.at[idx_d], av, sa).wait()
        pltpu.make_async_copy(b_hbm.at[idx_d], bv, sb).wait()
        pltpu.make_async_copy(c_hbm.at[pl.ds(0, _CH)], cv, semc0).wait()

        sh16 = jnp.full((_L,), 16, jnp.int32)
        msk = jnp.full((_L,), -65536, jnp.int32)

        @plsc.parallel_loop(0, _CH, unroll=4)
        def _(r):
            for j in range(_D // 32):
                v = cv[r, pl.ds(_L * j, _L)]
                c_lo = lax.bitcast_convert_type(
                    lax.shift_left(v, sh16), jnp.float32)
                c_hi = lax.bitcast_convert_type(
                    lax.bitwise_and(v, msk), jnp.float32)
                slo = pl.ds(32 * j, _L)
                shi = pl.ds(32 * j + _L, _L)
                av[r, slo] = jnp.maximum(av[r, slo] + bv[r, slo] + c_lo, 0.0)
                av[r, shi] = jnp.maximum(av[r, shi] + bv[r, shi] + c_hi, 0.0)

        pltpu.async_copy(
            c_hbm.at[pl.ds((tbase + jnp.minimum(g + 1, kc - 1)) * _CH,
                           _CH)], cv, semc0)
        pltpu.sync_copy(av, s_sh.at[idx_d], add=True)
        pltpu.async_copy(onesv, cnt_sh.at[idx_d], semcnt, add=True)

    def drain_cnt():
        for _ in range(_DBLK):
            pltpu.make_async_copy(onesv, cnt_sh.at[dblk0.at[0]],
                                  semcnt).wait()

    def block(b, cur, other):
        d_, s_ = cur
        do_, so_ = other

        @pl.when(b > 0)
        def _():
            drain_cnt()

        load_iblk(jnp.minimum(b + 1, nblk - 1), other)

        for j in range(_DBLK):
            g = b * _DBLK + j
            p = j % 2
            if j < _DBLK - 1:
                fetch(d_.at[j + 1], s_.at[j + 1], gsets[1 - p])
            else:
                fetch(do_.at[0], so_.at[0], gsets[1 - p])
            process(g, d_.at[j], gsets[p])

    load_iblk(0, iblks[0])
    fetch(dblk0.at[0], sblk0.at[0], gsets[0])
    pltpu.async_copy(c_hbm.at[pl.ds(tbase * _CH, _CH)], cv0, semc0)

    def super_body(i, carry):
        block(2 * i, iblks[0], iblks[1])
        block(2 * i + 1, iblks[1], iblks[0])
        return carry

    lax.fori_loop(0, nblk // 2, super_body, 0)

    av, bv, sa, sb = gsets[0]
    pltpu.make_async_copy(a_hbm.at[dblk0.at[0]], av, sa).wait()
    pltpu.make_async_copy(b_hbm.at[dblk0.at[0]], bv, sb).wait()
    pltpu.make_async_copy(c_hbm.at[pl.ds(0, _CH)], cv0, semc0).wait()
    drain_cnt()

    plsc.subcore_barrier()
    pltpu.sync_copy(s_sh.at[my_rows], out_s_hbm.at[cid, my_rows])

    @pl.when(sid == 0)
    def _():
        pltpu.sync_copy(cnt_sh, out_c_hbm.at[cid, 0])


def _build(interpret=False):
    node_pre = pl.pallas_call(
        _node_pre_body,
        grid=(_NS,),
        in_specs=[
            pl.BlockSpec((_RPS, _D), lambda i: (i, 0)),
            pl.BlockSpec((_D, 3 * _H), lambda i: (0, 0)),
        ],
        out_specs=[
            pl.BlockSpec((_RPS, _H), lambda i: (i, 0)),
            pl.BlockSpec((_RPS, _H), lambda i: (i, 0)),
            pl.BlockSpec((_RPS, _H), lambda i: (i, 0)),
        ],
        out_shape=[
            jax.ShapeDtypeStruct((_NP, _H), jnp.float32),
            jax.ShapeDtypeStruct((_NP, _H), jnp.float32),
            jax.ShapeDtypeStruct((_NP, _H), jnp.float32),
        ],
        interpret=interpret,
    )

    edge_pre = pl.pallas_call(
        _edge_pre_body,
        grid=(_EP // _EBLK,),
        in_specs=[
            pl.BlockSpec((_EBLK, _DE), lambda i: (i, 0)),
            pl.BlockSpec((_DE, _H), lambda i: (0, 0)),
            pl.BlockSpec((1, _H), lambda i: (0, 0)),
        ],
        out_specs=pl.BlockSpec((_EBLK, _H // 2), lambda i: (i, 0)),
        out_shape=jax.ShapeDtypeStruct((_EP, _H // 2), jnp.int32),
        interpret=interpret,
    )

    mesh = plsc.VectorSubcoreMesh(core_axis_name="c", subcore_axis_name="s")
    sc_edge = pl.kernel(
        _sc_edge_body,
        out_type=(
            jax.ShapeDtypeStruct((_NC, _NP, _D), jnp.float32),
            jax.ShapeDtypeStruct((_NC, 1, _NPC), jnp.float32),
        ),
        mesh=mesh,
        scratch_types=[
            pltpu.VMEM((_DBLK, _CH), jnp.int32),
            pltpu.VMEM((_DBLK, _CH), jnp.int32),
            pltpu.VMEM((_DBLK, _CH), jnp.int32),
            pltpu.VMEM((_DBLK, _CH), jnp.int32),
            pltpu.VMEM((_CH, _D), jnp.float32),
            pltpu.VMEM((_CH, _D), jnp.float32),
            pltpu.VMEM((_CH, _D), jnp.float32),
            pltpu.VMEM((_CH, _D), jnp.float32),
            pltpu.VMEM((_CH, _D // 2), jnp.int32),
            pltpu.VMEM((_CH,), jnp.float32),
            pltpu.VMEM_SHARED((_NP, _D), jnp.float32),
            pltpu.VMEM_SHARED((_NPC,), jnp.float32),
            pltpu.SemaphoreType.DMA,
            pltpu.SemaphoreType.DMA,
            pltpu.SemaphoreType.DMA,
            pltpu.SemaphoreType.DMA,
            pltpu.SemaphoreType.DMA,
            pltpu.SemaphoreType.DMA,
        ],
        interpret=interpret,
    )

    post = pl.pallas_call(
        _post_body,
        grid=(_N // _NBLK,),
        in_specs=[
            pl.BlockSpec((_NC, _NBLK, _D), lambda i: (0, i, 0)),
            pl.BlockSpec((_NBLK, _NC), lambda i: (i, 0)),
            pl.BlockSpec((_NBLK, _D), lambda i: (i, 0)),
            pl.BlockSpec((_D, _D), lambda i: (0, 0)),
            pl.BlockSpec((1, _D), lambda i: (0, 0)),
            pl.BlockSpec((1, _D), lambda i: (0, 0)),
            pl.BlockSpec((1, _D), lambda i: (0, 0)),
        ],
        out_specs=pl.BlockSpec((_NBLK, _D), lambda i: (i, 0)),
        out_shape=jax.ShapeDtypeStruct((_N, _D), jnp.float32),
        interpret=interpret,
    )

    @jax.jit
    def run(x, edge_index, edge_attr, W1, b1, W2, b2, Wskip, gamma, beta):
        dst = edge_index[0]
        src = edge_index[1]

        wn = jnp.concatenate([W1[:_D], W1[_D:2 * _D], Wskip], axis=1)
        a_tab, b_tab, skip = node_pre(x, wn)

        pad_dst = _N + (jnp.arange(_EP - _E, dtype=jnp.int32) % (_NP - _N))
        dst_p = jnp.concatenate([dst, pad_dst]).reshape(_EP // _CH, _CH)
        src_p = jnp.concatenate(
            [src, jnp.zeros((_EP - _E,), jnp.int32)]).reshape(
                _EP // _CH, _CH)
        ea_p = jnp.pad(edge_attr, ((0, _EP - _E), (0, 0)))
        w1e = W1[2 * _D:]
        colab = jnp.concatenate([jnp.asarray(_COLA), jnp.asarray(_COLB)])
        c_i32 = edge_pre(ea_p, w1e[:, colab], b1[colab].reshape(1, _H))

        z2 = jnp.zeros((_RPS, _D), jnp.float32)
        z1 = jnp.zeros((_CPS,), jnp.float32)
        s_out, cnt_out = sc_edge(dst_p, src_p, c_i32, a_tab, b_tab, z2, z1)

        cnt_t = jnp.swapaxes(cnt_out.reshape(_NC, _NPC), 0, 1)
        out = post(s_out, cnt_t, skip, W2, b2.reshape(1, _D),
                   gamma.reshape(1, _D), beta.reshape(1, _D))
        return out

    return run


_impl = _build()


def kernel(x, edge_index, edge_attr, W1, b1, W2, b2, Wskip, gamma, beta):
    return _impl(x, edge_index, edge_attr, W1, b1, W2, b2, Wskip, gamma,
                 beta)


# speedup vs baseline: 1.2879x; 1.0137x over previous
import functools

import jax
import jax.numpy as jnp
import numpy as np
from jax import lax
from jax.experimental import pallas as pl
from jax.experimental.pallas import tpu as pltpu
from jax.experimental.pallas import tpu_sc as plsc

_N = 10000
_E = 320000
_D = 128
_DE = 16
_H = 128

_NC = 2
_NS = 16
_NW = _NC * _NS
_L = 16

_CH = 64
_CPT = 160
_K0 = 208
_K1 = 2 * _CPT - _K0
_DBLK = 8
_EP = _NW * _CPT * _CH
_NP = 10112
_RPS = _NP // _NS
_NPC = 10240
_CPS = _NPC // _NS

_NBLK = 1000
_EBLK = 8192

_COLA = np.array([32 * (t // 16) + t % 16 for t in range(_D // 2)],
                 dtype=np.int32)
_COLB = _COLA + 16


def _node_pre_body(x_ref, w_ref, oa_ref, ob_ref, osk_ref):
    xw = jnp.dot(x_ref[...], w_ref[...],
                 preferred_element_type=jnp.float32)
    oa_ref[...] = xw[:, :_H]
    ob_ref[...] = xw[:, _H:2 * _H]
    osk_ref[...] = xw[:, 2 * _H:]


def _edge_pre_body(ea_ref, w_ref, b_ref, o_ref):
    o = jnp.dot(ea_ref[...], w_ref[...],
                preferred_element_type=jnp.float32) + b_ref[...]
    oa = o[:, :_H // 2]
    ob = o[:, _H // 2:]
    ua = lax.bitcast_convert_type(oa.astype(jnp.bfloat16),
                                  jnp.uint16).astype(jnp.uint32)
    ub = lax.bitcast_convert_type(ob.astype(jnp.bfloat16),
                                  jnp.uint16).astype(jnp.uint32)
    o_ref[...] = lax.bitcast_convert_type(ua | (ub << 16), jnp.int32)


def _post_body(s_ref, cnt_ref, skip_ref, w2_ref, b2_ref, g_ref, be_ref,
               o_ref):
    s = s_ref[0] + s_ref[1]
    cnt = cnt_ref[:, 0:1] + cnt_ref[:, 1:2]
    agg = jnp.dot(s, w2_ref[...], preferred_element_type=jnp.float32)
    agg = (agg + cnt * b2_ref[...]) / jnp.maximum(cnt, 1.0)
    out = agg + skip_ref[...]
    mu = jnp.mean(out, axis=-1, keepdims=True)
    var = jnp.mean((out - mu) ** 2, axis=-1, keepdims=True)
    o_ref[...] = (out - mu) * lax.rsqrt(var + 1e-5) * g_ref[...] + be_ref[...]


def _sc_edge_body(dst_hbm, src_hbm, c_hbm, a_hbm, b_hbm, z2_hbm, z1_hbm,
                  out_s_hbm, out_c_hbm,
                  dblk0, sblk0, dblk1, sblk1, av0, bv0, av1, bv1, cv0,
                  onesv, s_sh, cnt_sh,
                  sema0, semb0, sema1, semb1, semc0, semcnt):
    cid = lax.axis_index("c")
    sid = lax.axis_index("s")
    kc = lax.select(cid == 0, _K0, _K1)
    nblk = kc // _DBLK
    tbase = lax.select(cid == 0, sid * _K0, _NS * _K0 + sid * _K1)

    my_rows = pl.ds(sid * _RPS, _RPS)
    pltpu.sync_copy(z2_hbm, s_sh.at[my_rows])
    pltpu.sync_copy(z1_hbm, cnt_sh.at[pl.ds(sid * _CPS, _CPS)])

    for i in range(_CH // _L):
        onesv[pl.ds(i * _L, _L)] = jnp.full((_L,), 1.0, jnp.float32)

    plsc.subcore_barrier()

    gsets = ((av0, bv0, sema0, semb0),
             (av1, bv1, sema1, semb1))
    iblks = ((dblk0, sblk0), (dblk1, sblk1))

    def load_iblk(b, ib):
        d_, s_ = ib
        r0 = tbase + b * _DBLK
        pltpu.sync_copy(dst_hbm.at[pl.ds(r0, _DBLK)], d_)
        pltpu.sync_copy(src_hbm.at[pl.ds(r0, _DBLK)], s_)

    def fetch(idx_d, idx_s, gs):
        av, bv, sa, sb = gs
        pltpu.async_copy(a_hbm.at[idx_d], av, sa)
        pltpu.async_copy(b_hbm.at[idx_s], bv, sb)

    def process(g, idx_d, gs):
        av, bv, sa, sb = gs
        cv = cv0
        pltpu.make_async_copy(a_hbm.at[idx_d], av, sa).wait()
        pltpu.make_async_copy(b_hbm.at[idx_d], bv, sb).wait()
        pltpu.make_async_copy(c_hbm.at[pl.ds(0, _CH)], cv, semc0).wait()

        sh16 = jnp.full((_L,), 16, jnp.int32)
        msk = jnp.full((_L,), -65536, jnp.int32)

        @plsc.parallel_loop(0, _CH, unroll=4)
        def _(r):
            for j in range(_D // 32):
                v = cv[r, pl.ds(_L * j, _L)]
                c_lo = lax.bitcast_convert_type(
                    lax.shift_left(v, sh16), jnp.float32)
                c_hi = lax.bitcast_convert_type(
                    lax.bitwise_and(v, msk), jnp.float32)
                slo = pl.ds(32 * j, _L)
                shi = pl.ds(32 * j + _L, _L)
                av[r, slo] = jnp.maximum(av[r, slo] + bv[r, slo] + c_lo, 0.0)
                av[r, shi] = jnp.maximum(av[r, shi] + bv[r, shi] + c_hi, 0.0)

        pltpu.async_copy(
            c_hbm.at[pl.ds((tbase + jnp.minimum(g + 1, kc - 1)) * _CH,
                           _CH)], cv, semc0)
        pltpu.sync_copy(av, s_sh.at[idx_d], add=True)
        pltpu.async_copy(onesv, cnt_sh.at[idx_d], semcnt, add=True)

    def drain_cnt():
        for _ in range(_DBLK):
            pltpu.make_async_copy(onesv, cnt_sh.at[dblk0.at[0]],
                                  semcnt).wait()

    def block(b, cur, other):
        d_, s_ = cur
        do_, so_ = other

        @pl.when(b > 0)
        def _():
            drain_cnt()

        load_iblk(jnp.minimum(b + 1, nblk - 1), other)

        for j in range(_DBLK):
            g = b * _DBLK + j
            p = j % 2
            if j < _DBLK - 1:
                fetch(d_.at[j + 1], s_.at[j + 1], gsets[1 - p])
            else:
                fetch(do_.at[0], so_.at[0], gsets[1 - p])
            process(g, d_.at[j], gsets[p])

    load_iblk(0, iblks[0])
    fetch(dblk0.at[0], sblk0.at[0], gsets[0])
    pltpu.async_copy(c_hbm.at[pl.ds(tbase * _CH, _CH)], cv0, semc0)

    def super_body(i, carry):
        block(2 * i, iblks[0], iblks[1])
        block(2 * i + 1, iblks[1], iblks[0])
        return carry

    lax.fori_loop(0, nblk // 2, super_body, 0)

    av, bv, sa, sb = gsets[0]
    pltpu.make_async_copy(a_hbm.at[dblk0.at[0]], av, sa).wait()
    pltpu.make_async_copy(b_hbm.at[dblk0.at[0]], bv, sb).wait()
    pltpu.make_async_copy(c_hbm.at[pl.ds(0, _CH)], cv0, semc0).wait()
    drain_cnt()

    plsc.subcore_barrier()
    pltpu.sync_copy(s_sh.at[my_rows], out_s_hbm.at[cid, my_rows])

    @pl.when(sid == 0)
    def _():
        pltpu.sync_copy(cnt_sh, out_c_hbm.at[cid, 0])


def _build(interpret=False):
    node_pre = pl.pallas_call(
        _node_pre_body,
        grid=(_NS,),
        in_specs=[
            pl.BlockSpec((_RPS, _D), lambda i: (i, 0)),
            pl.BlockSpec((_D, 3 * _H), lambda i: (0, 0)),
        ],
        out_specs=[
            pl.BlockSpec((_RPS, _H), lambda i: (i, 0)),
            pl.BlockSpec((_RPS, _H), lambda i: (i, 0)),
            pl.BlockSpec((_RPS, _H), lambda i: (i, 0)),
        ],
        out_shape=[
            jax.ShapeDtypeStruct((_NP, _H), jnp.float32),
            jax.ShapeDtypeStruct((_NP, _H), jnp.float32),
            jax.ShapeDtypeStruct((_NP, _H), jnp.float32),
        ],
        interpret=interpret,
    )

    edge_pre = pl.pallas_call(
        _edge_pre_body,
        grid=(_EP // _EBLK,),
        in_specs=[
            pl.BlockSpec((_EBLK, _DE), lambda i: (i, 0)),
            pl.BlockSpec((_DE, _H), lambda i: (0, 0)),
            pl.BlockSpec((1, _H), lambda i: (0, 0)),
        ],
        out_specs=pl.BlockSpec((_EBLK, _H // 2), lambda i: (i, 0)),
        out_shape=jax.ShapeDtypeStruct((_EP, _H // 2), jnp.int32),
        interpret=interpret,
    )

    mesh = plsc.VectorSubcoreMesh(core_axis_name="c", subcore_axis_name="s")
    sc_edge = pl.kernel(
        _sc_edge_body,
        out_type=(
            jax.ShapeDtypeStruct((_NC, _NP, _D), jnp.float32),
            jax.ShapeDtypeStruct((_NC, 1, _NPC), jnp.float32),
        ),
        mesh=mesh,
        scratch_types=[
            pltpu.VMEM((_DBLK, _CH), jnp.int32),
            pltpu.VMEM((_DBLK, _CH), jnp.int32),
            pltpu.VMEM((_DBLK, _CH), jnp.int32),
            pltpu.VMEM((_DBLK, _CH), jnp.int32),
            pltpu.VMEM((_CH, _D), jnp.float32),
            pltpu.VMEM((_CH, _D), jnp.float32),
            pltpu.VMEM((_CH, _D), jnp.float32),
            pltpu.VMEM((_CH, _D), jnp.float32),
            pltpu.VMEM((_CH, _D // 2), jnp.int32),
            pltpu.VMEM((_CH,), jnp.float32),
            pltpu.VMEM_SHARED((_NP, _D), jnp.float32),
            pltpu.VMEM_SHARED((_NPC,), jnp.float32),
            pltpu.SemaphoreType.DMA,
            pltpu.SemaphoreType.DMA,
            pltpu.SemaphoreType.DMA,
            pltpu.SemaphoreType.DMA,
            pltpu.SemaphoreType.DMA,
            pltpu.SemaphoreType.DMA,
        ],
        interpret=interpret,
    )

    post = pl.pallas_call(
        _post_body,
        grid=(_N // _NBLK,),
        in_specs=[
            pl.BlockSpec((_NC, _NBLK, _D), lambda i: (0, i, 0)),
            pl.BlockSpec((_NBLK, _NC), lambda i: (i, 0)),
            pl.BlockSpec((_NBLK, _D), lambda i: (i, 0)),
            pl.BlockSpec((_D, _D), lambda i: (0, 0)),
            pl.BlockSpec((1, _D), lambda i: (0, 0)),
            pl.BlockSpec((1, _D), lambda i: (0, 0)),
            pl.BlockSpec((1, _D), lambda i: (0, 0)),
        ],
        out_specs=pl.BlockSpec((_NBLK, _D), lambda i: (i, 0)),
        out_shape=jax.ShapeDtypeStruct((_N, _D), jnp.float32),
        interpret=interpret,
    )

    @jax.jit
    def run(x, edge_index, edge_attr, W1, b1, W2, b2, Wskip, gamma, beta):
        dst = edge_index[0]
        src = edge_index[1]

        wn = jnp.concatenate([W1[:_D], W1[_D:2 * _D], Wskip], axis=1)
        a_tab, b_tab, skip = node_pre(x, wn)

        pad_dst = _N + (jnp.arange(_EP - _E, dtype=jnp.int32) % (_NP - _N))
        dst_p = jnp.concatenate([dst, pad_dst]).reshape(_EP // _CH, _CH)
        src_p = jnp.concatenate(
            [src, jnp.zeros((_EP - _E,), jnp.int32)]).reshape(
                _EP // _CH, _CH)
        ea_p = jnp.pad(edge_attr, ((0, _EP - _E), (0, 0)))
        w1e = W1[2 * _D:]
        colab = jnp.concatenate([jnp.asarray(_COLA), jnp.asarray(_COLB)])
        c_i32 = edge_pre(ea_p, w1e[:, colab], b1[colab].reshape(1, _H))

        z2 = jnp.zeros((_RPS, _D), jnp.float32)
        z1 = jnp.zeros((_CPS,), jnp.float32)
        s_out, cnt_out = sc_edge(dst_p, src_p, c_i32, a_tab, b_tab, z2, z1)

        cnt_t = jnp.swapaxes(cnt_out.reshape(_NC, _NPC), 0, 1)
        out = post(s_out, cnt_t, skip, W2, b2.reshape(1, _D),
                   gamma.reshape(1, _D), beta.reshape(1, _D))
        return out

    return run


_impl = _build()


def kernel(x, edge_index, edge_attr, W1, b1, W2, b2, Wskip, gamma, beta):
    return _impl(x, edge_index, edge_attr, W1, b1, W2, b2, Wskip, gamma,
                 beta)
